# baseline (speedup 1.0000x reference)
import functools

import jax
import jax.numpy as jnp
from jax import lax
from jax.experimental import pallas as pl
from jax.experimental.pallas import tpu as pltpu
from jax.experimental.pallas import tpu_sc as plsc

NCORES = 2
NSUB = 16
NW = NCORES * NSUB
LANES = 16
NUSER = 6000
NNODES = 10000
NPAD = 10240
D = 128
NEDGES = 320000
C = 128
K0 = 64
K1 = 96
KMAX = max(K0, K1)
EPAD = NSUB * (K0 + K1) * C
ROWS_T = NPAD // NSUB
DUMP = NNODES

_mesh = plsc.VectorSubcoreMesh(core_axis_name="c", subcore_axis_name="s")


@functools.partial(
    pl.kernel,
    out_type=jax.ShapeDtypeStruct((NCORES, NPAD, D), jnp.float32),
    mesh=_mesh,
    scratch_types=[
        pltpu.VMEM((KMAX, C), jnp.int32),
        pltpu.VMEM((2, C), jnp.int32),
        pltpu.VMEM((2, C), jnp.int32),
        pltpu.VMEM((KMAX * C,), jnp.float32),
        pltpu.VMEM((C, D), jnp.float32),
        pltpu.VMEM_SHARED((NPAD, D), jnp.float32),
        pltpu.SemaphoreType.DMA,
    ],
)
def _spmm_partials(x_hbm, packed_hbm, w_hbm, out_hbm,
                   packed_v, srcw, dstw, w_v, rows, acc_s, gsem):
    c = lax.axis_index("c")
    s = lax.axis_index("s")
    w_id = c * NSUB + s

    pltpu.sync_copy(packed_hbm.at[w_id], packed_v)
    pltpu.sync_copy(w_hbm.at[w_id], w_v)

    zero = jnp.zeros((LANES,), jnp.float32)

    @plsc.parallel_loop(0, C)
    def _zero_rows(e):
        row = rows.at[e]
        for d in range(D // LANES):
            row[pl.ds(d * LANES, LANES)] = zero

    for r in range(ROWS_T // C):
        pltpu.sync_copy(rows, acc_s.at[pl.ds(s * ROWS_T + r * C, C)])

    plsc.subcore_barrier()

    mask16 = jnp.full((LANES,), 0xFFFF, jnp.int32)

    def chunk_body(j, carry):
        @plsc.parallel_loop(0, C // LANES)
        def _unpack(g):
            sl = pl.ds(g * LANES, LANES)
            pk = packed_v.at[j][sl]
            dstw.at[0][sl] = lax.shift_right_logical(pk, 16)
            srcw.at[0][sl] = lax.bitwise_and(pk, mask16)

        pltpu.async_copy(x_hbm.at[srcw.at[0]], rows, gsem).wait()

        @plsc.parallel_loop(0, C // LANES)
        def _scale(g):
            wv = w_v[pl.ds(j * C + g * LANES, LANES)]
            for le in range(LANES):
                wspl = wv.at[jnp.full((LANES,), le, jnp.int32)].get(
                    mode="promise_in_bounds")
                row = rows.at[g * LANES + le]
                for d in range(D // LANES):
                    sl = pl.ds(d * LANES, LANES)
                    row[sl] = row[sl] * wspl

        pltpu.sync_copy(rows, acc_s.at[dstw.at[0]], add=True)
        return carry

    my_k = jnp.where(c == 0, K0, K1)
    lax.fori_loop(0, my_k, chunk_body, 0)

    plsc.subcore_barrier()

    for r in range(ROWS_T // C):
        base = s * ROWS_T + r * C
        pltpu.sync_copy(acc_s.at[pl.ds(base, C)], rows)
        pltpu.sync_copy(rows, out_hbm.at[c].at[pl.ds(base, C)])


_BLK = 512


def _sum_body(*refs):
    out = refs[-1]
    acc = refs[0][...]
    for r in refs[1:-1]:
        acc = acc + r[...]
    out[...] = acc


def _tc_sum(arrs):
    n = len(arrs)
    return pl.pallas_call(
        _sum_body,
        out_shape=jax.ShapeDtypeStruct((NPAD, D), jnp.float32),
        grid=(NPAD // _BLK,),
        in_specs=[pl.BlockSpec((_BLK, D), lambda i: (i, 0))] * n,
        out_specs=pl.BlockSpec((_BLK, D), lambda i: (i, 0)),
    )(*arrs)


def kernel(uEmbeds, iEmbeds, edge_weight, edge_index):
    x0 = jnp.concatenate([uEmbeds, iEmbeds], axis=0)
    x0p = jnp.pad(x0, ((0, NPAD - NNODES), (0, 0)))
    src = edge_index[1].astype(jnp.int32)
    dst = edge_index[0].astype(jnp.int32)
    w = edge_weight.astype(jnp.float32)

    packed_flat = jnp.bitwise_or(src, jnp.left_shift(dst, 16))
    pad = EPAD - NEDGES
    packed_flat = jnp.pad(packed_flat,
                          (0, pad), constant_values=DUMP << 16)
    w_flat = jnp.pad(w, (0, pad))
    n0 = NSUB * K0 * C

    def _slabs(a, fill):
        a0 = a[:n0].reshape(NSUB, K0, C)
        a0 = jnp.pad(a0, ((0, 0), (0, KMAX - K0), (0, 0)),
                     constant_values=fill)
        a1 = a[n0:].reshape(NSUB, K1, C)
        a1 = jnp.pad(a1, ((0, 0), (0, KMAX - K1), (0, 0)),
                     constant_values=fill)
        return jnp.concatenate([a0, a1], axis=0)

    packed = _slabs(packed_flat, DUMP << 16)
    w2 = _slabs(w_flat, 0.0).reshape(NW, KMAX * C)

    p = _spmm_partials(x0p, packed, w2)
    x1 = _tc_sum([p[0], p[1]])
    q = _spmm_partials(x1, packed, w2)
    out = _tc_sum([x0p, x1, q[0], q[1]])
    return (out[:NUSER], out[NUSER:NNODES])
